# Initial kernel scaffold; baseline (speedup 1.0000x reference)
import functools

import jax
import jax.numpy as jnp
from jax import lax
from jax.experimental import pallas as pl
from jax.experimental.pallas import tpu as pltpu
from jax.experimental.pallas import tpu_sc as plsc

F32 = jnp.float32
I32 = jnp.int32

NC = 2
NS = 16
NW = NC * NS
LANES = 16
EB = 128


def _sc_mesh():
    return plsc.VectorSubcoreMesh(
        core_axis_name="c", subcore_axis_name="s", num_cores=NC, num_subcores=NS
    )


def _make_sc_a(npad, nba, nbb, npc):
    @functools.partial(
        pl.kernel,
        out_type=(
            jax.ShapeDtypeStruct((npad,), F32),
            jax.ShapeDtypeStruct((NW, nbb, EB), F32),
        ),
        mesh=_sc_mesh(),
        scratch_types=[
            pltpu.VMEM_SHARED((npad,), F32),
            pltpu.VMEM((nba, EB), I32),
            pltpu.VMEM((nba, EB), F32),
            pltpu.VMEM((nbb, EB), I32),
            pltpu.VMEM((nbb, EB), F32),
            pltpu.VMEM((npad,), F32),
            pltpu.VMEM((npc,), F32),
        ],
    )
    def sc_a(srcA_hbm, ewA_hbm, srcB_hbm, ewB_hbm, z1_hbm,
             dinv_hbm, w2_hbm,
             deg_sh, srcA_v, ewA_v, srcB_v, ewB_v, dinv_f, deg_t):
        cid = lax.axis_index("c")
        sid = lax.axis_index("s")

        @pl.when(sid == 0)
        def _():
            pltpu.sync_copy(z1_hbm, deg_sh)

        pltpu.sync_copy(srcA_hbm.at[sid], srcA_v)
        pltpu.sync_copy(ewA_hbm.at[sid], ewA_v)
        plsc.subcore_barrier()

        def deg_body(j, carry):
            pltpu.sync_copy(ewA_v.at[j], deg_sh.at[srcA_v.at[j]], add=True)
            return carry

        lax.fori_loop(0, nba, deg_body, 0)
        plsc.subcore_barrier()

        pltpu.sync_copy(deg_sh.at[pl.ds(sid * npc, npc)], deg_t)

        def newton_body(k, carry):
            d = deg_t[pl.ds(k * LANES, LANES)]
            bits = plsc.bitcast(d, I32)
            y = plsc.bitcast(jnp.int32(0x5F3759DF) - (bits >> 1), F32)
            for _ in range(4):
                y = y * (1.5 - 0.5 * d * y * y)
            y = jnp.where(d > 0.0, y, 0.0)
            deg_t[pl.ds(k * LANES, LANES)] = y
            return carry

        lax.fori_loop(0, npc // LANES, newton_body, 0)
        pltpu.sync_copy(deg_t, deg_sh.at[pl.ds(sid * npc, npc)])

        @pl.when(cid == 0)
        def _():
            pltpu.sync_copy(deg_t, dinv_hbm.at[pl.ds(sid * npc, npc)])

        plsc.subcore_barrier()
        pltpu.sync_copy(deg_sh, dinv_f)

        wid = sid * NC + cid
        pltpu.sync_copy(srcB_hbm.at[wid], srcB_v)
        pltpu.sync_copy(ewB_hbm.at[wid], ewB_v)

        def w2_body(j, carry):
            for k in range(EB // LANES):
                sl = pl.ds(k * LANES, LANES)
                idx = srcB_v[j, sl]
                dv = plsc.load_gather(dinv_f, [idx])
                ewB_v[j, sl] = ewB_v[j, sl] * dv
            return carry

        lax.fori_loop(0, nbb, w2_body, 0)
        pltpu.sync_copy(ewB_v, w2_hbm.at[wid])

    return sc_a


def _make_sc_b(npad, nbb, npc, d):
    @functools.partial(
        pl.kernel,
        out_type=(
            jax.ShapeDtypeStruct((NC, npad, d), F32),
            jax.ShapeDtypeStruct((NC, npad, d), F32),
        ),
        mesh=_sc_mesh(),
        scratch_types=[
            pltpu.VMEM_SHARED((npad, d), F32),
            pltpu.VMEM((nbb, EB), I32),
            pltpu.VMEM((nbb, EB), I32),
            pltpu.VMEM((nbb, EB), F32),
            pltpu.VMEM((EB, d), F32),
            pltpu.VMEM((EB, d), F32),
            pltpu.VMEM((npc,), F32),
            pltpu.SemaphoreType.DMA,
            pltpu.SemaphoreType.DMA,
        ],
    )
    def sc_b(x_hbm, h_hbm, src_hbm, dst_hbm, w2_hbm, dinv_hbm, z2_hbm,
             s_out, t_out,
             acc, src_v, dst_v, w2_v, bufA, bufB, dinv_t, semA, semB):
        cid = lax.axis_index("c")
        sid = lax.axis_index("s")
        wid = sid * NC + cid
        rs = sid * npc

        pltpu.sync_copy(src_hbm.at[wid], src_v)
        pltpu.sync_copy(dst_hbm.at[wid], dst_v)
        pltpu.sync_copy(z2_hbm.at[pl.ds(rs, npc)], acc.at[pl.ds(rs, npc)])
        plsc.subcore_barrier()

        def s_body(j, carry):
            pltpu.async_copy(x_hbm.at[dst_v.at[j]], bufA, semA).wait()
            pltpu.sync_copy(bufA, acc.at[src_v.at[j]], add=True)
            pltpu.async_copy(x_hbm.at[src_v.at[j]], bufB, semB).wait()
            pltpu.sync_copy(bufB, acc.at[dst_v.at[j]], add=True)
            return carry

        lax.fori_loop(0, nbb, s_body, 0)
        plsc.subcore_barrier()
        pltpu.sync_copy(acc.at[pl.ds(rs, npc)], s_out.at[cid, pl.ds(rs, npc)])
        plsc.subcore_barrier()

        pltpu.sync_copy(z2_hbm.at[pl.ds(rs, npc)], acc.at[pl.ds(rs, npc)])
        pltpu.sync_copy(w2_hbm.at[wid], w2_v)
        plsc.subcore_barrier()

        def t_body(j, carry):
            pltpu.async_copy(h_hbm.at[src_v.at[j]], bufA, semA).wait()

            def scale_body(e, c2):
                sv = jnp.full((LANES,), w2_v[j, e], F32)
                for k in range(d // LANES):
                    sl = pl.ds(k * LANES, LANES)
                    bufA[e, sl] = bufA[e, sl] * sv
                return c2

            lax.fori_loop(0, EB, scale_body, 0)
            pltpu.sync_copy(bufA, acc.at[dst_v.at[j]], add=True)
            return carry

        lax.fori_loop(0, nbb, t_body, 0)
        plsc.subcore_barrier()

        pltpu.sync_copy(dinv_hbm.at[pl.ds(rs, npc)], dinv_t)

        def out_body(q, carry):
            base = rs + q * EB
            pltpu.sync_copy(acc.at[pl.ds(base, EB)], bufB)

            def row_body(r, c2):
                sv = jnp.full((LANES,), -dinv_t[q * EB + r], F32)
                for k in range(d // LANES):
                    sl = pl.ds(k * LANES, LANES)
                    bufB[r, sl] = bufB[r, sl] * sv
                return c2

            lax.fori_loop(0, EB, row_body, 0)
            pltpu.sync_copy(bufB, t_out.at[cid, pl.ds(base, EB)])
            return carry

        lax.fori_loop(0, npc // EB, out_body, 0)

    return sc_b


def _tc_body(x_ref, h_ref, c_ref, sp_ref, tp_ref,
             wg_ref, t0_ref, t1_ref, bg_ref, lw_ref, lb_ref,
             out_ref, h0_ref, c0_ref):
    hid = h_ref.shape[1]
    x1 = x_ref[...] + sp_ref[0] + sp_ref[1]
    tx1 = tp_ref[0] + tp_ref[1]
    hcur = h_ref[...]
    z = (
        jnp.dot(x1, wg_ref[...], preferred_element_type=F32,
                precision=lax.Precision.HIGHEST)
        + jnp.dot(hcur, t0_ref[...], preferred_element_type=F32,
                  precision=lax.Precision.HIGHEST)
        + jnp.dot(tx1, t1_ref[...], preferred_element_type=F32,
                  precision=lax.Precision.HIGHEST)
        + bg_ref[...]
    )
    gi = jax.nn.sigmoid(z[:, :hid])
    gf = jax.nn.sigmoid(z[:, hid:2 * hid])
    gt = jnp.tanh(z[:, 2 * hid:3 * hid])
    go = jax.nn.sigmoid(z[:, 3 * hid:])
    c0 = gf * c_ref[...] + gi * gt
    h0 = go * jnp.tanh(c0)
    out = jnp.dot(jnp.maximum(h0, 0.0), lw_ref[...], preferred_element_type=F32,
                  precision=lax.Precision.HIGHEST) + lb_ref[...]
    out_ref[...] = out
    h0_ref[...] = h0
    c0_ref[...] = c0


def kernel(x, edge_index, edge_weight, h, c, snapshot_ts, params):
    n, d = x.shape
    hid = h.shape[1]
    e = edge_index.shape[1]

    npad = ((n + 1 + NW * EB - 1) // (NW * EB)) * NW * EB
    npc = npad // NS
    ep = ((e + NW * EB - 1) // (NW * EB)) * NW * EB
    nbb = ep // (NW * EB)
    nba = ep // (NS * EB)

    src = edge_index[0]
    dst = edge_index[1]
    padn = jnp.full((ep - e,), n, I32)
    srcp = jnp.concatenate([src, padn])
    dstp = jnp.concatenate([dst, padn])
    ewp = jnp.concatenate([edge_weight, jnp.zeros((ep - e,), F32)])

    srcB = srcp.reshape(NW, nbb, EB)
    dstB = dstp.reshape(NW, nbb, EB)
    ewB = ewp.reshape(NW, nbb, EB)
    srcA = srcp.reshape(NS, nba, EB)
    ewA = ewp.reshape(NS, nba, EB)

    x_pad = jnp.zeros((npad, d), F32).at[:n].set(x)
    h_pad = jnp.zeros((npad, hid), F32).at[:n].set(h)
    z1 = jnp.zeros((npad,), F32)
    z2 = jnp.zeros((npad, d), F32)

    sc_a = _make_sc_a(npad, nba, nbb, npc)
    dinv, w2 = sc_a(srcA, ewA, srcB, ewB, z1)

    sc_b = _make_sc_b(npad, nbb, npc, d)
    s_parts, t_parts = sc_b(x_pad, h_pad, srcB, dstB, w2, dinv, z2)

    wg = jnp.concatenate([params["W_" + g] for g in ("i", "f", "c", "o")], axis=1)
    t0g = jnp.concatenate([params["T0_" + g] for g in ("i", "f", "c", "o")], axis=1)
    t1g = jnp.concatenate([params["T1_" + g] for g in ("i", "f", "c", "o")], axis=1)
    bg = jnp.concatenate(
        [params["b_" + g] + params["bc_" + g][None, :] for g in ("i", "f", "c", "o")],
        axis=1)
    lw = params["lin_W"]
    lb = params["lin_b"][None, :]

    rows = 1000
    grid = (n // rows,)
    sds = jax.ShapeDtypeStruct
    out, h0, c0 = pl.pallas_call(
        _tc_body,
        grid=grid,
        in_specs=[
            pl.BlockSpec((rows, d), lambda i: (i, 0)),
            pl.BlockSpec((rows, hid), lambda i: (i, 0)),
            pl.BlockSpec((rows, hid), lambda i: (i, 0)),
            pl.BlockSpec((NC, rows, d), lambda i: (0, i, 0)),
            pl.BlockSpec((NC, rows, hid), lambda i: (0, i, 0)),
            pl.BlockSpec((d, 4 * hid), lambda i: (0, 0)),
            pl.BlockSpec((hid, 4 * hid), lambda i: (0, 0)),
            pl.BlockSpec((hid, 4 * hid), lambda i: (0, 0)),
            pl.BlockSpec((1, 4 * hid), lambda i: (0, 0)),
            pl.BlockSpec((hid, hid), lambda i: (0, 0)),
            pl.BlockSpec((1, hid), lambda i: (0, 0)),
        ],
        out_specs=[
            pl.BlockSpec((rows, hid), lambda i: (i, 0)),
            pl.BlockSpec((rows, hid), lambda i: (i, 0)),
            pl.BlockSpec((rows, hid), lambda i: (i, 0)),
        ],
        out_shape=(
            sds((n, hid), F32),
            sds((n, hid), F32),
            sds((n, hid), F32),
        ),
    )(x, h, c, s_parts, t_parts, wg, t0g, t1g, bg, lw, lb)

    return (out, h0, c0)


# speedup vs baseline: 7.3281x; 7.3281x over previous
import functools

import jax
import jax.numpy as jnp
from jax import lax
from jax.experimental import pallas as pl
from jax.experimental.pallas import tpu as pltpu
from jax.experimental.pallas import tpu_sc as plsc

F32 = jnp.float32
I32 = jnp.int32

NC = 2
NS = 16
NW = NC * NS
LANES = 16
EB = 128


def _sc_mesh():
    return plsc.VectorSubcoreMesh(
        core_axis_name="c", subcore_axis_name="s", num_cores=NC, num_subcores=NS
    )


def _make_sc_a(npad, nba, nbb, npc):
    @functools.partial(
        pl.kernel,
        out_type=(
            jax.ShapeDtypeStruct((npad,), F32),
            jax.ShapeDtypeStruct((NW, nbb, EB), F32),
        ),
        mesh=_sc_mesh(),
        scratch_types=[
            pltpu.VMEM_SHARED((npad,), F32),
            pltpu.VMEM((nba, EB), I32),
            pltpu.VMEM((nba, EB), F32),
            pltpu.VMEM((nbb, EB), I32),
            pltpu.VMEM((nbb, EB), F32),
            pltpu.VMEM((EB,), F32),
            pltpu.VMEM((npc,), F32),
        ],
    )
    def sc_a(srcA_hbm, ewA_hbm, srcB_hbm, ewB_hbm, z1_hbm,
             dinv_hbm, w2_hbm,
             deg_sh, srcA_v, ewA_v, srcB_v, ewB_v, dv_buf, deg_t):
        cid = lax.axis_index("c")
        sid = lax.axis_index("s")

        @pl.when(sid == 0)
        def _():
            pltpu.sync_copy(z1_hbm, deg_sh)

        pltpu.sync_copy(srcA_hbm.at[sid], srcA_v)
        pltpu.sync_copy(ewA_hbm.at[sid], ewA_v)
        plsc.subcore_barrier()

        def deg_body(j, carry):
            pltpu.sync_copy(ewA_v.at[j], deg_sh.at[srcA_v.at[j]], add=True)
            return carry

        lax.fori_loop(0, nba, deg_body, 0)
        plsc.subcore_barrier()

        pltpu.sync_copy(deg_sh.at[pl.ds(sid * npc, npc)], deg_t)

        def newton_body(k, carry):
            d = deg_t[pl.ds(k * LANES, LANES)]
            bits = lax.bitcast_convert_type(d, I32)
            y = lax.bitcast_convert_type(jnp.int32(0x5F3759DF) - (bits >> 1), F32)
            for _ in range(4):
                y = y * (1.5 - 0.5 * d * y * y)
            y = jnp.where(d > 0.0, y, 0.0)
            deg_t[pl.ds(k * LANES, LANES)] = y
            return carry

        lax.fori_loop(0, npc // LANES, newton_body, 0)
        pltpu.sync_copy(deg_t, deg_sh.at[pl.ds(sid * npc, npc)])

        @pl.when(cid == 0)
        def _():
            pltpu.sync_copy(deg_t, dinv_hbm.at[pl.ds(sid * npc, npc)])

        plsc.subcore_barrier()

        wid = sid * NC + cid
        pltpu.sync_copy(srcB_hbm.at[wid], srcB_v)
        pltpu.sync_copy(ewB_hbm.at[wid], ewB_v)

        def w2_body(j, carry):
            pltpu.sync_copy(deg_sh.at[srcB_v.at[j]], dv_buf)
            for k in range(EB // LANES):
                sl = pl.ds(k * LANES, LANES)
                ewB_v[j, sl] = ewB_v[j, sl] * dv_buf[sl]
            return carry

        lax.fori_loop(0, nbb, w2_body, 0)
        pltpu.sync_copy(ewB_v, w2_hbm.at[wid])

    return sc_a


def _make_sc_b(npad, nbb, npc, d):
    @functools.partial(
        pl.kernel,
        out_type=(
            jax.ShapeDtypeStruct((NC, npad, d), F32),
            jax.ShapeDtypeStruct((NC, npad, d), F32),
        ),
        mesh=_sc_mesh(),
        scratch_types=[
            pltpu.VMEM_SHARED((npad, d), F32),
            pltpu.VMEM((nbb, EB), I32),
            pltpu.VMEM((nbb, EB), I32),
            pltpu.VMEM((EB,), F32),
            pltpu.VMEM((EB, d), F32),
            pltpu.VMEM((npc,), F32),
            pltpu.SemaphoreType.DMA,
            pltpu.SemaphoreType.DMA,
        ],
    )
    def sc_b(x_hbm, h_hbm, src_hbm, dst_hbm, w2_hbm, dinv_hbm, z2_hbm,
             s_out, t_out,
             acc, src_v, dst_v, w2b, bufA, dinv_t, semA, semB):
        cid = lax.axis_index("c")
        sid = lax.axis_index("s")
        wid = sid * NC + cid
        rs = sid * npc

        pltpu.sync_copy(src_hbm.at[wid], src_v)
        pltpu.sync_copy(dst_hbm.at[wid], dst_v)
        pltpu.sync_copy(z2_hbm.at[pl.ds(rs, npc)], acc.at[pl.ds(rs, npc)])
        plsc.subcore_barrier()

        def s_body(j, carry):
            pltpu.async_copy(x_hbm.at[dst_v.at[j]], bufA, semA).wait()
            pltpu.sync_copy(bufA, acc.at[src_v.at[j]], add=True)
            pltpu.async_copy(x_hbm.at[src_v.at[j]], bufA, semB).wait()
            pltpu.sync_copy(bufA, acc.at[dst_v.at[j]], add=True)
            return carry

        lax.fori_loop(0, nbb, s_body, 0)
        plsc.subcore_barrier()
        pltpu.sync_copy(acc.at[pl.ds(rs, npc)], s_out.at[cid, pl.ds(rs, npc)])
        plsc.subcore_barrier()

        pltpu.sync_copy(z2_hbm.at[pl.ds(rs, npc)], acc.at[pl.ds(rs, npc)])
        plsc.subcore_barrier()

        def t_body(j, carry):
            pltpu.sync_copy(w2_hbm.at[wid].at[j], w2b)
            pltpu.async_copy(h_hbm.at[src_v.at[j]], bufA, semA).wait()

            def scale_body(g, c2):
                wv = w2b[pl.ds(g * LANES, LANES)]
                for l in range(LANES):
                    sv = jnp.full((LANES,), wv[l], F32)
                    e2 = g * LANES + l
                    for k in range(d // LANES):
                        sl = pl.ds(k * LANES, LANES)
                        bufA[e2, sl] = bufA[e2, sl] * sv
                return c2

            lax.fori_loop(0, EB // LANES, scale_body, 0)
            pltpu.sync_copy(bufA, acc.at[dst_v.at[j]], add=True)
            return carry

        lax.fori_loop(0, nbb, t_body, 0)
        plsc.subcore_barrier()

        pltpu.sync_copy(dinv_hbm.at[pl.ds(rs, npc)], dinv_t)

        def out_body(q, carry):
            base = rs + q * EB
            pltpu.sync_copy(acc.at[pl.ds(base, EB)], bufA)

            def row_body(g, c2):
                dv = dinv_t[pl.ds(q * EB + g * LANES, LANES)]
                for l in range(LANES):
                    sv = jnp.full((LANES,), -dv[l], F32)
                    r = g * LANES + l
                    for k in range(d // LANES):
                        sl = pl.ds(k * LANES, LANES)
                        bufA[r, sl] = bufA[r, sl] * sv
                return c2

            lax.fori_loop(0, EB // LANES, row_body, 0)
            pltpu.sync_copy(bufA, t_out.at[cid, pl.ds(base, EB)])
            return carry

        lax.fori_loop(0, npc // EB, out_body, 0)

    return sc_b


def _tc_body(x_ref, h_ref, c_ref, sp_ref, tp_ref,
             wg_ref, t0_ref, t1_ref, bg_ref, lw_ref, lb_ref,
             out_ref, h0_ref, c0_ref):
    hid = h_ref.shape[1]
    x1 = x_ref[...] + sp_ref[0] + sp_ref[1]
    tx1 = tp_ref[0] + tp_ref[1]
    hcur = h_ref[...]
    z = (
        jnp.dot(x1, wg_ref[...], preferred_element_type=F32,
                precision=lax.Precision.HIGHEST)
        + jnp.dot(hcur, t0_ref[...], preferred_element_type=F32,
                  precision=lax.Precision.HIGHEST)
        + jnp.dot(tx1, t1_ref[...], preferred_element_type=F32,
                  precision=lax.Precision.HIGHEST)
        + bg_ref[...]
    )
    gi = jax.nn.sigmoid(z[:, :hid])
    gf = jax.nn.sigmoid(z[:, hid:2 * hid])
    gt = jnp.tanh(z[:, 2 * hid:3 * hid])
    go = jax.nn.sigmoid(z[:, 3 * hid:])
    c0 = gf * c_ref[...] + gi * gt
    h0 = go * jnp.tanh(c0)
    out = jnp.dot(jnp.maximum(h0, 0.0), lw_ref[...], preferred_element_type=F32,
                  precision=lax.Precision.HIGHEST) + lb_ref[...]
    out_ref[...] = out
    h0_ref[...] = h0
    c0_ref[...] = c0


def kernel(x, edge_index, edge_weight, h, c, snapshot_ts, params):
    n, d = x.shape
    hid = h.shape[1]
    e = edge_index.shape[1]

    npad = ((n + 1 + NS * EB - 1) // (NS * EB)) * NS * EB
    npc = npad // NS
    ep = ((e + NW * EB - 1) // (NW * EB)) * NW * EB
    nbb = ep // (NW * EB)
    nba = ep // (NS * EB)

    src = edge_index[0]
    dst = edge_index[1]
    padn = jnp.full((ep - e,), n, I32)
    srcp = jnp.concatenate([src, padn])
    dstp = jnp.concatenate([dst, padn])
    ewp = jnp.concatenate([edge_weight, jnp.zeros((ep - e,), F32)])

    srcB = srcp.reshape(NW, nbb, EB)
    dstB = dstp.reshape(NW, nbb, EB)
    ewB = ewp.reshape(NW, nbb, EB)
    srcA = srcp.reshape(NS, nba, EB)
    ewA = ewp.reshape(NS, nba, EB)

    x_pad = jnp.zeros((npad, d), F32).at[:n].set(x)
    h_pad = jnp.zeros((npad, hid), F32).at[:n].set(h)
    z1 = jnp.zeros((npad,), F32)
    z2 = jnp.zeros((npad, d), F32)

    sc_a = _make_sc_a(npad, nba, nbb, npc)
    dinv, w2 = sc_a(srcA, ewA, srcB, ewB, z1)

    sc_b = _make_sc_b(npad, nbb, npc, d)
    s_parts, t_parts = sc_b(x_pad, h_pad, srcB, dstB, w2, dinv, z2)

    wg = jnp.concatenate([params["W_" + g] for g in ("i", "f", "c", "o")], axis=1)
    t0g = jnp.concatenate([params["T0_" + g] for g in ("i", "f", "c", "o")], axis=1)
    t1g = jnp.concatenate([params["T1_" + g] for g in ("i", "f", "c", "o")], axis=1)
    bg = jnp.concatenate(
        [params["b_" + g] + params["bc_" + g][None, :] for g in ("i", "f", "c", "o")],
        axis=1)
    lw = params["lin_W"]
    lb = params["lin_b"][None, :]

    rows = 1000
    grid = (n // rows,)
    sds = jax.ShapeDtypeStruct
    out, h0, c0 = pl.pallas_call(
        _tc_body,
        grid=grid,
        in_specs=[
            pl.BlockSpec((rows, d), lambda i: (i, 0)),
            pl.BlockSpec((rows, hid), lambda i: (i, 0)),
            pl.BlockSpec((rows, hid), lambda i: (i, 0)),
            pl.BlockSpec((NC, rows, d), lambda i: (0, i, 0)),
            pl.BlockSpec((NC, rows, hid), lambda i: (0, i, 0)),
            pl.BlockSpec((d, 4 * hid), lambda i: (0, 0)),
            pl.BlockSpec((hid, 4 * hid), lambda i: (0, 0)),
            pl.BlockSpec((hid, 4 * hid), lambda i: (0, 0)),
            pl.BlockSpec((1, 4 * hid), lambda i: (0, 0)),
            pl.BlockSpec((hid, hid), lambda i: (0, 0)),
            pl.BlockSpec((1, hid), lambda i: (0, 0)),
        ],
        out_specs=[
            pl.BlockSpec((rows, hid), lambda i: (i, 0)),
            pl.BlockSpec((rows, hid), lambda i: (i, 0)),
            pl.BlockSpec((rows, hid), lambda i: (i, 0)),
        ],
        out_shape=(
            sds((n, hid), F32),
            sds((n, hid), F32),
            sds((n, hid), F32),
        ),
    )(x, h, c, s_parts, t_parts, wg, t0g, t1g, bg, lw, lb)

    return (out, h0, c0)
